# baseline (speedup 1.0000x reference)
import jax
import jax.numpy as jnp
from jax import lax
from jax.experimental import pallas as pl
from jax.experimental.pallas import tpu as pltpu
from jax.experimental.pallas import tpu_sc as plsc

N_NODES = 10000
N_EDGES = 320000
D_NODE = 128
D_EDGE = 16
MSG = 32

NC = 2
NS = 16
NW = NC * NS

C_EDGES = 1024
CQ = C_EDGES // 4
SUB = 128
R_CHUNK = C_EDGES // SUB
N_CHUNK = 10
EPW = C_EDGES * N_CHUNK
N_EDGES_PAD = NW * EPW
N_IDX_ROWS = N_EDGES_PAD // SUB
N_NODES_PAD = 10016
NPZ = 624


def _proj_nodes_body(x_ref, w0_ref, w1_ref, p0_ref, p1_ref):
    x = x_ref[...]
    zt = jnp.zeros((N_NODES_PAD - N_NODES, MSG), jnp.float32)
    p0_ref[pl.ds(0, N_NODES), :] = jnp.dot(
        x, w0_ref[...], preferred_element_type=jnp.float32
    )
    p0_ref[pl.ds(N_NODES, N_NODES_PAD - N_NODES), :] = zt
    p1_ref[pl.ds(0, N_NODES), :] = jnp.dot(
        x, w1_ref[...], preferred_element_type=jnp.float32
    )
    p1_ref[pl.ds(N_NODES, N_NODES_PAD - N_NODES), :] = zt


def _proj_nodes(x, w0T, w1T):
    return pl.pallas_call(
        _proj_nodes_body,
        out_shape=[
            jax.ShapeDtypeStruct((N_NODES_PAD, MSG), jnp.float32),
            jax.ShapeDtypeStruct((N_NODES_PAD, MSG), jnp.float32),
        ],
    )(x, w0T, w1T)


def _proj_edges_body(efT_ref, wc_ref, b1_ref, ep_ref):
    res = (
        lax.dot_general(
            efT_ref[...], wc_ref[...],
            (((0,), (0,)), ((), ())),
            preferred_element_type=jnp.float32,
        )
        + b1_ref[...]
    )
    ep_ref[...] = jnp.concatenate(
        [res[0:256], res[256:512], res[512:768], res[768:1024]], axis=1
    )


def _proj_edges(efTp, wcT, b1r):
    grid = N_EDGES_PAD // C_EDGES
    return pl.pallas_call(
        _proj_edges_body,
        grid=(grid,),
        in_specs=[
            pl.BlockSpec((D_EDGE, C_EDGES), lambda i: (0, i)),
            pl.BlockSpec((D_EDGE, MSG), lambda i: (0, 0)),
            pl.BlockSpec((1, MSG), lambda i: (0, 0)),
        ],
        out_specs=pl.BlockSpec((CQ, 4 * MSG), lambda i: (i, 0)),
        out_shape=jax.ShapeDtypeStruct((N_EDGES_PAD // 4, 4 * MSG), jnp.float32),
    )(efTp, wcT, b1r)


def _sc_body(p0_hbm, p1_hbm, ep_hbm, i0_hbm, i1_hbm, out_hbm,
             i0_v, i1_v, ep_v, g0_v, g1_v, acc_sh, sem):
    cid = lax.axis_index("c")
    sid = lax.axis_index("s")
    wid = sid * NC + cid

    def zrow(r, carry):
        g0_v[r, pl.ds(0, 16)] = jnp.zeros((16,), jnp.float32)
        g0_v[r, pl.ds(16, 16)] = jnp.zeros((16,), jnp.float32)
        return carry

    lax.fori_loop(0, NPZ + 32, zrow, 0)
    pltpu.sync_copy(g0_v.at[pl.ds(0, NPZ)], acc_sh.at[pl.ds(sid * NPZ, NPZ)])

    @pl.when(sid == NS - 1)
    def _zero_tail():
        pltpu.sync_copy(
            g0_v.at[pl.ds(0, 32)], acc_sh.at[pl.ds(NS * NPZ, 32)]
        )

    plsc.subcore_barrier()

    def chunk(ci, carry):
        jc = wid * N_CHUNK + ci
        rbase = jc * R_CHUNK
        pltpu.sync_copy(i0_hbm.at[pl.ds(rbase, R_CHUNK)], i0_v)
        pltpu.sync_copy(i1_hbm.at[pl.ds(rbase, R_CHUNK)], i1_v)
        cps = [pltpu.async_copy(ep_hbm.at[pl.ds(jc * CQ, CQ)], ep_v, sem)]
        for j in range(R_CHUNK):
            dst = pl.ds(j * SUB, SUB)
            cps.append(pltpu.async_copy(p0_hbm.at[i0_v.at[j]], g0_v.at[dst], sem))
            cps.append(pltpu.async_copy(p1_hbm.at[i1_v.at[j]], g1_v.at[dst], sem))
        for c in cps:
            c.wait()

        def rowf(q, rcarry):
            r4 = q * 4
            for u in range(4):
                for off in (0, 16):
                    s = pl.ds(off, 16)
                    se = pl.ds(32 * u + off, 16)
                    g0_v[r4 + u, s] = jnp.maximum(
                        g0_v[r4 + u, s] + g1_v[r4 + u, s] + ep_v[q, se], 0.0
                    )
            return rcarry

        lax.fori_loop(0, CQ, rowf, 0)
        for j in range(R_CHUNK):
            pltpu.sync_copy(
                g0_v.at[pl.ds(j * SUB, SUB)], acc_sh.at[i0_v.at[j]], add=True
            )
        return carry

    lax.fori_loop(0, N_CHUNK, chunk, 0)
    plsc.subcore_barrier()
    pltpu.sync_copy(
        acc_sh.at[pl.ds(sid * NPZ, NPZ)], out_hbm.at[cid, pl.ds(sid * NPZ, NPZ)]
    )

    @pl.when(sid == NS - 1)
    def _write_tail():
        pltpu.sync_copy(
            acc_sh.at[pl.ds(NS * NPZ, 16)], out_hbm.at[cid, pl.ds(NS * NPZ, 16)]
        )


def _sc_gather_scatter(P0, P1, EP, i0, i1):
    mesh = plsc.VectorSubcoreMesh(core_axis_name="c", subcore_axis_name="s")
    return pl.kernel(
        _sc_body,
        out_type=jax.ShapeDtypeStruct((NC, N_NODES, MSG), jnp.float32),
        mesh=mesh,
        compiler_params=pltpu.CompilerParams(use_tc_tiling_on_sc=False),
        scratch_types=[
            pltpu.VMEM((R_CHUNK, SUB), jnp.int32),
            pltpu.VMEM((R_CHUNK, SUB), jnp.int32),
            pltpu.VMEM((CQ, 4 * MSG), jnp.float32),
            pltpu.VMEM((C_EDGES, MSG), jnp.float32),
            pltpu.VMEM((C_EDGES, MSG), jnp.float32),
            pltpu.VMEM_SHARED((N_NODES_PAD, MSG), jnp.float32),
            pltpu.SemaphoreType.DMA,
        ],
    )(P0, P1, EP, i0, i1)


def _final_body(x_ref, part_ref, w2a_ref, w2b_ref, b2_ref, out_ref):
    acc = part_ref[0] + part_ref[1]
    o = (
        jnp.dot(x_ref[...], w2a_ref[...], preferred_element_type=jnp.float32)
        + jnp.dot(acc, w2b_ref[...], preferred_element_type=jnp.float32)
        + b2_ref[...]
    )
    out_ref[...] = jnp.maximum(o, 0.0)


def _final(x, part, w2aT, w2bT, b2r):
    blk = 1000
    grid = N_NODES // blk
    return pl.pallas_call(
        _final_body,
        grid=(grid,),
        in_specs=[
            pl.BlockSpec((blk, D_NODE), lambda i: (i, 0)),
            pl.BlockSpec((NC, blk, MSG), lambda i: (0, i, 0)),
            pl.BlockSpec((D_NODE, D_NODE), lambda i: (0, 0)),
            pl.BlockSpec((MSG, D_NODE), lambda i: (0, 0)),
            pl.BlockSpec((1, D_NODE), lambda i: (0, 0)),
        ],
        out_specs=pl.BlockSpec((blk, D_NODE), lambda i: (i, 0)),
        out_shape=jax.ShapeDtypeStruct((N_NODES, D_NODE), jnp.float32),
    )(x, part, w2aT, w2bT, b2r)


def kernel(node_features, edge_node_indices, edge_features, W1, b1, W2, b2):
    x = node_features
    n0 = edge_node_indices[0].astype(jnp.int32)
    n1 = edge_node_indices[1].astype(jnp.int32)
    w0T = W1[:, :D_NODE].T
    w1T = W1[:, D_NODE:2 * D_NODE].T
    wcT = W1[:, 2 * D_NODE:].T
    w2aT = W2[:, :D_NODE].T
    w2bT = W2[:, D_NODE:].T
    b1r = b1.reshape(1, MSG)
    b2r = b2.reshape(1, D_NODE)

    P0, P1 = _proj_nodes(x, w0T, w1T)

    efTp = jnp.pad(edge_features.T, ((0, 0), (0, N_EDGES_PAD - N_EDGES)))
    EP = _proj_edges(efTp, wcT, b1r)

    def _prep_idx(idx):
        a = jnp.pad(
            idx, (0, N_EDGES_PAD - N_EDGES), constant_values=N_NODES
        )
        a = a.reshape(N_EDGES_PAD // C_EDGES, 4, CQ).transpose(0, 2, 1)
        return a.reshape(N_IDX_ROWS, SUB)

    part = _sc_gather_scatter(P0, P1, EP, _prep_idx(n0), _prep_idx(n1))
    return _final(x, part, w2aT, w2bT, b2r)


# speedup vs baseline: 1.5205x; 1.5205x over previous
import jax
import jax.numpy as jnp
from jax import lax
from jax.experimental import pallas as pl
from jax.experimental.pallas import tpu as pltpu
from jax.experimental.pallas import tpu_sc as plsc

N_NODES = 10000
N_EDGES = 320000
D_NODE = 128
D_EDGE = 16
MSG = 32

NC = 2
NS = 16
NW = NC * NS

C_EDGES = 1024
CQ = C_EDGES // 4
SUB = 128
R_CHUNK = C_EDGES // SUB
N_CHUNK = 10
EPW = C_EDGES * N_CHUNK
N_EDGES_PAD = NW * EPW
N_IDX_ROWS = N_EDGES_PAD // SUB
N_NODES_PAD = 10016
NPZ = 624


def _proj_nodes_body(x_ref, w0_ref, w1_ref, p0_ref, p1_ref):
    x = x_ref[...]
    zt = jnp.zeros((N_NODES_PAD - N_NODES, MSG), jnp.float32)
    p0_ref[pl.ds(0, N_NODES), :] = jnp.dot(
        x, w0_ref[...], preferred_element_type=jnp.float32
    )
    p0_ref[pl.ds(N_NODES, N_NODES_PAD - N_NODES), :] = zt
    p1_ref[pl.ds(0, N_NODES), :] = jnp.dot(
        x, w1_ref[...], preferred_element_type=jnp.float32
    )
    p1_ref[pl.ds(N_NODES, N_NODES_PAD - N_NODES), :] = zt


def _proj_nodes(x, w0T, w1T):
    return pl.pallas_call(
        _proj_nodes_body,
        out_shape=[
            jax.ShapeDtypeStruct((N_NODES_PAD, MSG), jnp.float32),
            jax.ShapeDtypeStruct((N_NODES_PAD, MSG), jnp.float32),
        ],
    )(x, w0T, w1T)


E_BLK = 4 * C_EDGES


def _proj_edges_body(efT_ref, wc_ref, b1_ref, ep_ref):
    res = (
        lax.dot_general(
            efT_ref[...], wc_ref[...],
            (((0,), (0,)), ((), ())),
            preferred_element_type=jnp.float32,
        )
        + b1_ref[...]
    )
    packed = [
        jnp.concatenate(
            [res[1024 * t + 256 * u:1024 * t + 256 * u + 256] for u in range(4)],
            axis=1,
        )
        for t in range(4)
    ]
    ep_ref[...] = jnp.concatenate(packed, axis=0)


def _proj_edges(efTp, wcT, b1r):
    grid = N_EDGES_PAD // E_BLK
    return pl.pallas_call(
        _proj_edges_body,
        grid=(grid,),
        in_specs=[
            pl.BlockSpec((D_EDGE, E_BLK), lambda i: (0, i)),
            pl.BlockSpec((D_EDGE, MSG), lambda i: (0, 0)),
            pl.BlockSpec((1, MSG), lambda i: (0, 0)),
        ],
        out_specs=pl.BlockSpec((E_BLK // 4, 4 * MSG), lambda i: (i, 0)),
        out_shape=jax.ShapeDtypeStruct((N_EDGES_PAD // 4, 4 * MSG), jnp.float32),
    )(efTp, wcT, b1r)


def _sc_body(p0_hbm, p1_hbm, ep_hbm, i0_hbm, i1_hbm, out_hbm,
             i0_v, i1_v, ep_v, g0_v, g1_v, acc_sh, sem):
    cid = lax.axis_index("c")
    sid = lax.axis_index("s")
    wid = sid * NC + cid

    def zrow(r, carry):
        g0_v[r, pl.ds(0, 16)] = jnp.zeros((16,), jnp.float32)
        g0_v[r, pl.ds(16, 16)] = jnp.zeros((16,), jnp.float32)
        return carry

    lax.fori_loop(0, NPZ + 32, zrow, 0)
    pltpu.sync_copy(g0_v.at[pl.ds(0, NPZ)], acc_sh.at[pl.ds(sid * NPZ, NPZ)])

    @pl.when(sid == NS - 1)
    def _zero_tail():
        pltpu.sync_copy(
            g0_v.at[pl.ds(0, 32)], acc_sh.at[pl.ds(NS * NPZ, 32)]
        )

    plsc.subcore_barrier()

    def chunk(ci, carry):
        jc = wid * N_CHUNK + ci
        rbase = jc * R_CHUNK
        pltpu.sync_copy(i0_hbm.at[pl.ds(rbase, R_CHUNK)], i0_v)
        pltpu.sync_copy(i1_hbm.at[pl.ds(rbase, R_CHUNK)], i1_v)
        cps = [pltpu.async_copy(ep_hbm.at[pl.ds(jc * CQ, CQ)], ep_v, sem)]
        for j in range(R_CHUNK):
            dst = pl.ds(j * SUB, SUB)
            cps.append(pltpu.async_copy(p0_hbm.at[i0_v.at[j]], g0_v.at[dst], sem))
            cps.append(pltpu.async_copy(p1_hbm.at[i1_v.at[j]], g1_v.at[dst], sem))
        for c in cps:
            c.wait()

        def rowf(q, rcarry):
            for u in range(4):
                for off in (0, 16):
                    s = pl.ds(off, 16)
                    se = pl.ds(32 * u + off, 16)
                    g0_v[q + CQ * u, s] = jnp.maximum(
                        g0_v[q + CQ * u, s] + g1_v[q + CQ * u, s] + ep_v[q, se],
                        0.0,
                    )
            return rcarry

        lax.fori_loop(0, CQ, rowf, 0)
        for j in range(R_CHUNK):
            pltpu.sync_copy(
                g0_v.at[pl.ds(j * SUB, SUB)], acc_sh.at[i0_v.at[j]], add=True
            )
        return carry

    lax.fori_loop(0, N_CHUNK, chunk, 0)
    plsc.subcore_barrier()
    pltpu.sync_copy(
        acc_sh.at[pl.ds(sid * NPZ, NPZ)], out_hbm.at[cid, pl.ds(sid * NPZ, NPZ)]
    )

    @pl.when(sid == NS - 1)
    def _write_tail():
        pltpu.sync_copy(
            acc_sh.at[pl.ds(NS * NPZ, 16)], out_hbm.at[cid, pl.ds(NS * NPZ, 16)]
        )


def _sc_gather_scatter(P0, P1, EP, i0, i1):
    mesh = plsc.VectorSubcoreMesh(core_axis_name="c", subcore_axis_name="s")
    return pl.kernel(
        _sc_body,
        out_type=jax.ShapeDtypeStruct((NC, N_NODES, MSG), jnp.float32),
        mesh=mesh,
        compiler_params=pltpu.CompilerParams(use_tc_tiling_on_sc=False),
        scratch_types=[
            pltpu.VMEM((R_CHUNK, SUB), jnp.int32),
            pltpu.VMEM((R_CHUNK, SUB), jnp.int32),
            pltpu.VMEM((CQ, 4 * MSG), jnp.float32),
            pltpu.VMEM((C_EDGES, MSG), jnp.float32),
            pltpu.VMEM((C_EDGES, MSG), jnp.float32),
            pltpu.VMEM_SHARED((N_NODES_PAD, MSG), jnp.float32),
            pltpu.SemaphoreType.DMA,
        ],
    )(P0, P1, EP, i0, i1)


def _final_body(x_ref, part_ref, w2a_ref, w2b_ref, b2_ref, out_ref):
    acc = part_ref[0] + part_ref[1]
    o = (
        jnp.dot(x_ref[...], w2a_ref[...], preferred_element_type=jnp.float32)
        + jnp.dot(acc, w2b_ref[...], preferred_element_type=jnp.float32)
        + b2_ref[...]
    )
    out_ref[...] = jnp.maximum(o, 0.0)


def _final(x, part, w2aT, w2bT, b2r):
    blk = 1000
    grid = N_NODES // blk
    return pl.pallas_call(
        _final_body,
        grid=(grid,),
        in_specs=[
            pl.BlockSpec((blk, D_NODE), lambda i: (i, 0)),
            pl.BlockSpec((NC, blk, MSG), lambda i: (0, i, 0)),
            pl.BlockSpec((D_NODE, D_NODE), lambda i: (0, 0)),
            pl.BlockSpec((MSG, D_NODE), lambda i: (0, 0)),
            pl.BlockSpec((1, D_NODE), lambda i: (0, 0)),
        ],
        out_specs=pl.BlockSpec((blk, D_NODE), lambda i: (i, 0)),
        out_shape=jax.ShapeDtypeStruct((N_NODES, D_NODE), jnp.float32),
    )(x, part, w2aT, w2bT, b2r)


def kernel(node_features, edge_node_indices, edge_features, W1, b1, W2, b2):
    x = node_features
    n0 = edge_node_indices[0].astype(jnp.int32)
    n1 = edge_node_indices[1].astype(jnp.int32)
    w0T = W1[:, :D_NODE].T
    w1T = W1[:, D_NODE:2 * D_NODE].T
    wcT = W1[:, 2 * D_NODE:].T
    w2aT = W2[:, :D_NODE].T
    w2bT = W2[:, D_NODE:].T
    b1r = b1.reshape(1, MSG)
    b2r = b2.reshape(1, D_NODE)

    P0, P1 = _proj_nodes(x, w0T, w1T)

    efTp = jnp.pad(edge_features.T, ((0, 0), (0, N_EDGES_PAD - N_EDGES)))
    EP = _proj_edges(efTp, wcT, b1r)

    trash = N_NODES + jnp.arange(N_EDGES_PAD - N_EDGES, dtype=jnp.int32) % 16

    def _prep_idx(idx):
        a = jnp.concatenate([idx, trash])
        return a.reshape(N_IDX_ROWS, SUB)

    part = _sc_gather_scatter(P0, P1, EP, _prep_idx(n0), _prep_idx(n1))
    return _final(x, part, w2aT, w2bT, b2r)


# speedup vs baseline: 1.9321x; 1.2707x over previous
import jax
import jax.numpy as jnp
from jax import lax
from jax.experimental import pallas as pl
from jax.experimental.pallas import tpu as pltpu
from jax.experimental.pallas import tpu_sc as plsc

N_NODES = 10000
N_EDGES = 320000
D_NODE = 128
D_EDGE = 16
MSG = 32

NC = 2
NS = 16
NW = NC * NS

C_EDGES = 512
CQ = C_EDGES // 4
SUB = 128
R_CHUNK = C_EDGES // SUB
N_CHUNK = 20
EPW = C_EDGES * N_CHUNK
N_EDGES_PAD = NW * EPW
N_IDX_ROWS = N_EDGES_PAD // SUB
N_NODES_PAD = 10016
NPZ = 624


def _proj_nodes_body(x_ref, w0_ref, w1_ref, p0_ref, p1_ref):
    x = x_ref[...]
    zt = jnp.zeros((N_NODES_PAD - N_NODES, MSG), jnp.float32)
    p0_ref[pl.ds(0, N_NODES), :] = jnp.dot(
        x, w0_ref[...], preferred_element_type=jnp.float32
    )
    p0_ref[pl.ds(N_NODES, N_NODES_PAD - N_NODES), :] = zt
    p1_ref[pl.ds(0, N_NODES), :] = jnp.dot(
        x, w1_ref[...], preferred_element_type=jnp.float32
    )
    p1_ref[pl.ds(N_NODES, N_NODES_PAD - N_NODES), :] = zt


def _proj_nodes(x, w0T, w1T):
    return pl.pallas_call(
        _proj_nodes_body,
        out_shape=[
            jax.ShapeDtypeStruct((N_NODES_PAD, MSG), jnp.float32),
            jax.ShapeDtypeStruct((N_NODES_PAD, MSG), jnp.float32),
        ],
    )(x, w0T, w1T)


E_BLK = 4096


def _proj_edges_body(efT_ref, wc_ref, b1_ref, ep_ref):
    res = (
        lax.dot_general(
            efT_ref[...], wc_ref[...],
            (((0,), (0,)), ((), ())),
            preferred_element_type=jnp.float32,
        )
        + b1_ref[...]
    )
    packed = [
        jnp.concatenate(
            [
                res[C_EDGES * t + CQ * u:C_EDGES * t + CQ * (u + 1)]
                for u in range(4)
            ],
            axis=1,
        )
        for t in range(E_BLK // C_EDGES)
    ]
    ep_ref[...] = jnp.concatenate(packed, axis=0)


def _proj_edges(efTp, wcT, b1r):
    grid = N_EDGES_PAD // E_BLK
    return pl.pallas_call(
        _proj_edges_body,
        grid=(grid,),
        in_specs=[
            pl.BlockSpec((D_EDGE, E_BLK), lambda i: (0, i)),
            pl.BlockSpec((D_EDGE, MSG), lambda i: (0, 0)),
            pl.BlockSpec((1, MSG), lambda i: (0, 0)),
        ],
        out_specs=pl.BlockSpec((E_BLK // 4, 4 * MSG), lambda i: (i, 0)),
        out_shape=jax.ShapeDtypeStruct((N_EDGES_PAD // 4, 4 * MSG), jnp.float32),
    )(efTp, wcT, b1r)


def _sc_body(p0_hbm, p1_hbm, ep_hbm, i0_hbm, i1_hbm, out_hbm,
             i0_v0, i0_v1, i1_v0, i1_v1, ep_v0, ep_v1,
             g0_v0, g0_v1, g1_v0, g1_v1, acc_sh, sem0, sem1):
    cid = lax.axis_index("c")
    sid = lax.axis_index("s")
    wid = sid * NC + cid
    i0_v = (i0_v0, i0_v1)
    i1_v = (i1_v0, i1_v1)
    ep_v = (ep_v0, ep_v1)
    g0_v = (g0_v0, g0_v1)
    g1_v = (g1_v0, g1_v1)
    sem = (sem0, sem1)

    def zrow(r, carry):
        g0_v0[r, pl.ds(0, 16)] = jnp.zeros((16,), jnp.float32)
        g0_v0[r, pl.ds(16, 16)] = jnp.zeros((16,), jnp.float32)
        return carry

    lax.fori_loop(0, C_EDGES, zrow, 0)
    pltpu.sync_copy(
        g0_v0, acc_sh.at[pl.ds(sid * NPZ, C_EDGES)]
    )
    pltpu.sync_copy(
        g0_v0.at[pl.ds(0, NPZ - C_EDGES)],
        acc_sh.at[pl.ds(sid * NPZ + C_EDGES, NPZ - C_EDGES)],
    )

    @pl.when(sid == NS - 1)
    def _zero_tail():
        pltpu.sync_copy(
            g0_v0.at[pl.ds(0, 32)], acc_sh.at[pl.ds(NS * NPZ, 32)]
        )

    plsc.subcore_barrier()

    def _start(slot, jc):
        rbase = jc * R_CHUNK
        pltpu.sync_copy(i0_hbm.at[pl.ds(rbase, R_CHUNK)], i0_v[slot])
        pltpu.sync_copy(i1_hbm.at[pl.ds(rbase, R_CHUNK)], i1_v[slot])
        pltpu.async_copy(ep_hbm.at[pl.ds(jc * CQ, CQ)], ep_v[slot], sem[slot])
        for j in range(R_CHUNK):
            dst = pl.ds(j * SUB, SUB)
            pltpu.async_copy(p0_hbm.at[i0_v[slot].at[j]], g0_v[slot].at[dst],
                             sem[slot])
            pltpu.async_copy(p1_hbm.at[i1_v[slot].at[j]], g1_v[slot].at[dst],
                             sem[slot])

    def _drain(slot, jc):
        pltpu.make_async_copy(
            ep_hbm.at[pl.ds(jc * CQ, CQ)], ep_v[slot], sem[slot]
        ).wait()
        for j in range(R_CHUNK):
            dst = pl.ds(j * SUB, SUB)
            pltpu.make_async_copy(
                p0_hbm.at[i0_v[slot].at[j]], g0_v[slot].at[dst], sem[slot]
            ).wait()
            pltpu.make_async_copy(
                p1_hbm.at[i1_v[slot].at[j]], g1_v[slot].at[dst], sem[slot]
            ).wait()

    def _process(slot, jc):
        _drain(slot, jc)

        def rowf(q, rcarry):
            for u in range(4):
                for off in (0, 16):
                    s = pl.ds(off, 16)
                    se = pl.ds(32 * u + off, 16)
                    g0_v[slot][q + CQ * u, s] = jnp.maximum(
                        g0_v[slot][q + CQ * u, s]
                        + g1_v[slot][q + CQ * u, s]
                        + ep_v[slot][q, se],
                        0.0,
                    )
            return rcarry

        lax.fori_loop(0, CQ, rowf, 0)
        for j in range(R_CHUNK):
            pltpu.sync_copy(
                g0_v[slot].at[pl.ds(j * SUB, SUB)],
                acc_sh.at[i0_v[slot].at[j]],
                add=True,
            )

    jc0 = wid * N_CHUNK
    _start(0, jc0)

    def pair(pi, carry):
        jc_a = jc0 + 2 * pi
        _start(1, jc_a + 1)
        _process(0, jc_a)

        @pl.when(pi < N_CHUNK // 2 - 1)
        def _next():
            _start(0, jc_a + 2)

        _process(1, jc_a + 1)
        return carry

    lax.fori_loop(0, N_CHUNK // 2, pair, 0)
    plsc.subcore_barrier()
    pltpu.sync_copy(
        acc_sh.at[pl.ds(sid * NPZ, NPZ)], out_hbm.at[cid, pl.ds(sid * NPZ, NPZ)]
    )

    @pl.when(sid == NS - 1)
    def _write_tail():
        pltpu.sync_copy(
            acc_sh.at[pl.ds(NS * NPZ, 16)], out_hbm.at[cid, pl.ds(NS * NPZ, 16)]
        )


def _sc_gather_scatter(P0, P1, EP, i0, i1):
    mesh = plsc.VectorSubcoreMesh(core_axis_name="c", subcore_axis_name="s")
    return pl.kernel(
        _sc_body,
        out_type=jax.ShapeDtypeStruct((NC, N_NODES, MSG), jnp.float32),
        mesh=mesh,
        compiler_params=pltpu.CompilerParams(use_tc_tiling_on_sc=False),
        scratch_types=[
            pltpu.VMEM((R_CHUNK, SUB), jnp.int32),
            pltpu.VMEM((R_CHUNK, SUB), jnp.int32),
            pltpu.VMEM((R_CHUNK, SUB), jnp.int32),
            pltpu.VMEM((R_CHUNK, SUB), jnp.int32),
            pltpu.VMEM((CQ, 4 * MSG), jnp.float32),
            pltpu.VMEM((CQ, 4 * MSG), jnp.float32),
            pltpu.VMEM((C_EDGES, MSG), jnp.float32),
            pltpu.VMEM((C_EDGES, MSG), jnp.float32),
            pltpu.VMEM((C_EDGES, MSG), jnp.float32),
            pltpu.VMEM((C_EDGES, MSG), jnp.float32),
            pltpu.VMEM_SHARED((N_NODES_PAD, MSG), jnp.float32),
            pltpu.SemaphoreType.DMA,
            pltpu.SemaphoreType.DMA,
        ],
    )(P0, P1, EP, i0, i1)


def _final_body(x_ref, part_ref, w2a_ref, w2b_ref, b2_ref, out_ref):
    acc = part_ref[0] + part_ref[1]
    o = (
        jnp.dot(x_ref[...], w2a_ref[...], preferred_element_type=jnp.float32)
        + jnp.dot(acc, w2b_ref[...], preferred_element_type=jnp.float32)
        + b2_ref[...]
    )
    out_ref[...] = jnp.maximum(o, 0.0)


def _final(x, part, w2aT, w2bT, b2r):
    blk = 1000
    grid = N_NODES // blk
    return pl.pallas_call(
        _final_body,
        grid=(grid,),
        in_specs=[
            pl.BlockSpec((blk, D_NODE), lambda i: (i, 0)),
            pl.BlockSpec((NC, blk, MSG), lambda i: (0, i, 0)),
            pl.BlockSpec((D_NODE, D_NODE), lambda i: (0, 0)),
            pl.BlockSpec((MSG, D_NODE), lambda i: (0, 0)),
            pl.BlockSpec((1, D_NODE), lambda i: (0, 0)),
        ],
        out_specs=pl.BlockSpec((blk, D_NODE), lambda i: (i, 0)),
        out_shape=jax.ShapeDtypeStruct((N_NODES, D_NODE), jnp.float32),
    )(x, part, w2aT, w2bT, b2r)


def kernel(node_features, edge_node_indices, edge_features, W1, b1, W2, b2):
    x = node_features
    n0 = edge_node_indices[0].astype(jnp.int32)
    n1 = edge_node_indices[1].astype(jnp.int32)
    w0T = W1[:, :D_NODE].T
    w1T = W1[:, D_NODE:2 * D_NODE].T
    wcT = W1[:, 2 * D_NODE:].T
    w2aT = W2[:, :D_NODE].T
    w2bT = W2[:, D_NODE:].T
    b1r = b1.reshape(1, MSG)
    b2r = b2.reshape(1, D_NODE)

    P0, P1 = _proj_nodes(x, w0T, w1T)

    efTp = jnp.pad(edge_features.T, ((0, 0), (0, N_EDGES_PAD - N_EDGES)))
    EP = _proj_edges(efTp, wcT, b1r)

    trash = N_NODES + jnp.arange(N_EDGES_PAD - N_EDGES, dtype=jnp.int32) % 16

    def _prep_idx(idx):
        a = jnp.concatenate([idx, trash])
        return a.reshape(N_IDX_ROWS, SUB)

    part = _sc_gather_scatter(P0, P1, EP, _prep_idx(n0), _prep_idx(n1))
    return _final(x, part, w2aT, w2bT, b2r)


# speedup vs baseline: 2.2186x; 1.1483x over previous
import jax
import jax.numpy as jnp
from jax import lax
from jax.experimental import pallas as pl
from jax.experimental.pallas import tpu as pltpu
from jax.experimental.pallas import tpu_sc as plsc

N_NODES = 10000
N_EDGES = 320000
D_NODE = 128
D_EDGE = 16
MSG = 32

NC = 2
NS = 16
NW = NC * NS

C_EDGES = 512
CQ = C_EDGES // 4
SUB = 128
R_CHUNK = C_EDGES // SUB
N_CHUNK = 20
EPW = C_EDGES * N_CHUNK
N_EDGES_PAD = NW * EPW
N_IDX_ROWS = N_EDGES_PAD // SUB
N_NODES_PAD = 10016
NPZ = 624


def _proj_nodes_body(x_ref, w0_ref, w1_ref, p0_ref, p1_ref):
    x = x_ref[...]
    zt = jnp.zeros((N_NODES_PAD - N_NODES, MSG), jnp.float32)
    p0_ref[pl.ds(0, N_NODES), :] = jnp.dot(
        x, w0_ref[...], preferred_element_type=jnp.float32
    )
    p0_ref[pl.ds(N_NODES, N_NODES_PAD - N_NODES), :] = zt
    p1_ref[pl.ds(0, N_NODES), :] = jnp.dot(
        x, w1_ref[...], preferred_element_type=jnp.float32
    )
    p1_ref[pl.ds(N_NODES, N_NODES_PAD - N_NODES), :] = zt


def _proj_nodes(x, w0T, w1T):
    return pl.pallas_call(
        _proj_nodes_body,
        out_shape=[
            jax.ShapeDtypeStruct((N_NODES_PAD, MSG), jnp.float32),
            jax.ShapeDtypeStruct((N_NODES_PAD, MSG), jnp.float32),
        ],
    )(x, w0T, w1T)


E_BLK = 4096


def _proj_edges_body(efT_ref, wc_ref, b1_ref, ep_ref):
    res = (
        lax.dot_general(
            efT_ref[...], wc_ref[...],
            (((0,), (0,)), ((), ())),
            preferred_element_type=jnp.float32,
        )
        + b1_ref[...]
    )
    packed = [
        jnp.concatenate(
            [
                res[C_EDGES * t + CQ * u:C_EDGES * t + CQ * (u + 1)]
                for u in range(4)
            ],
            axis=1,
        )
        for t in range(E_BLK // C_EDGES)
    ]
    ep_ref[...] = jnp.concatenate(packed, axis=0)


def _proj_edges(efTp, wcT, b1r):
    grid = N_EDGES_PAD // E_BLK
    return pl.pallas_call(
        _proj_edges_body,
        grid=(grid,),
        in_specs=[
            pl.BlockSpec((D_EDGE, E_BLK), lambda i: (0, i)),
            pl.BlockSpec((D_EDGE, MSG), lambda i: (0, 0)),
            pl.BlockSpec((1, MSG), lambda i: (0, 0)),
        ],
        out_specs=pl.BlockSpec((E_BLK // 4, 4 * MSG), lambda i: (i, 0)),
        out_shape=jax.ShapeDtypeStruct((N_EDGES_PAD // 4, 4 * MSG), jnp.float32),
    )(efTp, wcT, b1r)


def _sc_body(p0_hbm, p1_hbm, ep_hbm, i0_hbm, i1_hbm, out_hbm,
             i0_v0, i0_v1, i1_v0, i1_v1, ep_v0, ep_v1,
             g0_v0, g0_v1, g1_v0, g1_v1, acc_sh, sem0, sem1):
    cid = lax.axis_index("c")
    sid = lax.axis_index("s")
    wid = sid * NC + cid
    i0_v = (i0_v0, i0_v1)
    i1_v = (i1_v0, i1_v1)
    ep_v = (ep_v0, ep_v1)
    g0_v = (g0_v0, g0_v1)
    g1_v = (g1_v0, g1_v1)
    sem = (sem0, sem1)

    def zrow(r, carry):
        g0_v0[r, pl.ds(0, 16)] = jnp.zeros((16,), jnp.float32)
        g0_v0[r, pl.ds(16, 16)] = jnp.zeros((16,), jnp.float32)
        return carry

    lax.fori_loop(0, C_EDGES, zrow, 0)
    pltpu.sync_copy(
        g0_v0, acc_sh.at[pl.ds(sid * NPZ, C_EDGES)]
    )
    pltpu.sync_copy(
        g0_v0.at[pl.ds(0, NPZ - C_EDGES)],
        acc_sh.at[pl.ds(sid * NPZ + C_EDGES, NPZ - C_EDGES)],
    )

    @pl.when(sid == NS - 1)
    def _zero_tail():
        pltpu.sync_copy(
            g0_v0.at[pl.ds(0, 32)], acc_sh.at[pl.ds(NS * NPZ, 32)]
        )

    plsc.subcore_barrier()

    def _start(slot, jc):
        rbase = jc * R_CHUNK
        pltpu.sync_copy(i0_hbm.at[pl.ds(rbase, R_CHUNK)], i0_v[slot])
        pltpu.sync_copy(i1_hbm.at[pl.ds(rbase, R_CHUNK)], i1_v[slot])
        pltpu.async_copy(ep_hbm.at[pl.ds(jc * CQ, CQ)], ep_v[slot], sem[slot])
        for j in range(R_CHUNK):
            dst = pl.ds(j * SUB, SUB)
            pltpu.async_copy(p0_hbm.at[i0_v[slot].at[j]], g0_v[slot].at[dst],
                             sem[slot])
            pltpu.async_copy(p1_hbm.at[i1_v[slot].at[j]], g1_v[slot].at[dst],
                             sem[slot])

    def _drain(slot, jc):
        pltpu.make_async_copy(
            ep_hbm.at[pl.ds(jc * CQ, CQ)], ep_v[slot], sem[slot]
        ).wait()
        for j in range(R_CHUNK):
            dst = pl.ds(j * SUB, SUB)
            pltpu.make_async_copy(
                p0_hbm.at[i0_v[slot].at[j]], g0_v[slot].at[dst], sem[slot]
            ).wait()
            pltpu.make_async_copy(
                p1_hbm.at[i1_v[slot].at[j]], g1_v[slot].at[dst], sem[slot]
            ).wait()

    def _process(slot, jc):
        _drain(slot, jc)

        @plsc.parallel_loop(0, CQ, unroll=2)
        def rowf(q):
            for u in range(4):
                for off in (0, 16):
                    s = pl.ds(off, 16)
                    se = pl.ds(32 * u + off, 16)
                    g0_v[slot][q + CQ * u, s] = jnp.maximum(
                        g0_v[slot][q + CQ * u, s]
                        + g1_v[slot][q + CQ * u, s]
                        + ep_v[slot][q, se],
                        0.0,
                    )
        for j in range(R_CHUNK):
            pltpu.sync_copy(
                g0_v[slot].at[pl.ds(j * SUB, SUB)],
                acc_sh.at[i0_v[slot].at[j]],
                add=True,
            )

    jc0 = wid * N_CHUNK
    _start(0, jc0)

    def pair(pi, carry):
        jc_a = jc0 + 2 * pi
        _start(1, jc_a + 1)
        _process(0, jc_a)

        @pl.when(pi < N_CHUNK // 2 - 1)
        def _next():
            _start(0, jc_a + 2)

        _process(1, jc_a + 1)
        return carry

    lax.fori_loop(0, N_CHUNK // 2, pair, 0)
    plsc.subcore_barrier()
    pltpu.sync_copy(
        acc_sh.at[pl.ds(sid * NPZ, NPZ)], out_hbm.at[cid, pl.ds(sid * NPZ, NPZ)]
    )

    @pl.when(sid == NS - 1)
    def _write_tail():
        pltpu.sync_copy(
            acc_sh.at[pl.ds(NS * NPZ, 16)], out_hbm.at[cid, pl.ds(NS * NPZ, 16)]
        )


def _sc_gather_scatter(P0, P1, EP, i0, i1):
    mesh = plsc.VectorSubcoreMesh(core_axis_name="c", subcore_axis_name="s")
    return pl.kernel(
        _sc_body,
        out_type=jax.ShapeDtypeStruct((NC, N_NODES, MSG), jnp.float32),
        mesh=mesh,
        compiler_params=pltpu.CompilerParams(use_tc_tiling_on_sc=False),
        scratch_types=[
            pltpu.VMEM((R_CHUNK, SUB), jnp.int32),
            pltpu.VMEM((R_CHUNK, SUB), jnp.int32),
            pltpu.VMEM((R_CHUNK, SUB), jnp.int32),
            pltpu.VMEM((R_CHUNK, SUB), jnp.int32),
            pltpu.VMEM((CQ, 4 * MSG), jnp.float32),
            pltpu.VMEM((CQ, 4 * MSG), jnp.float32),
            pltpu.VMEM((C_EDGES, MSG), jnp.float32),
            pltpu.VMEM((C_EDGES, MSG), jnp.float32),
            pltpu.VMEM((C_EDGES, MSG), jnp.float32),
            pltpu.VMEM((C_EDGES, MSG), jnp.float32),
            pltpu.VMEM_SHARED((N_NODES_PAD, MSG), jnp.float32),
            pltpu.SemaphoreType.DMA,
            pltpu.SemaphoreType.DMA,
        ],
    )(P0, P1, EP, i0, i1)


def _final_body(x_ref, part_ref, w2a_ref, w2b_ref, b2_ref, out_ref):
    acc = part_ref[0] + part_ref[1]
    o = (
        jnp.dot(x_ref[...], w2a_ref[...], preferred_element_type=jnp.float32)
        + jnp.dot(acc, w2b_ref[...], preferred_element_type=jnp.float32)
        + b2_ref[...]
    )
    out_ref[...] = jnp.maximum(o, 0.0)


def _final(x, part, w2aT, w2bT, b2r):
    blk = 1000
    grid = N_NODES // blk
    return pl.pallas_call(
        _final_body,
        grid=(grid,),
        in_specs=[
            pl.BlockSpec((blk, D_NODE), lambda i: (i, 0)),
            pl.BlockSpec((NC, blk, MSG), lambda i: (0, i, 0)),
            pl.BlockSpec((D_NODE, D_NODE), lambda i: (0, 0)),
            pl.BlockSpec((MSG, D_NODE), lambda i: (0, 0)),
            pl.BlockSpec((1, D_NODE), lambda i: (0, 0)),
        ],
        out_specs=pl.BlockSpec((blk, D_NODE), lambda i: (i, 0)),
        out_shape=jax.ShapeDtypeStruct((N_NODES, D_NODE), jnp.float32),
    )(x, part, w2aT, w2bT, b2r)


def kernel(node_features, edge_node_indices, edge_features, W1, b1, W2, b2):
    x = node_features
    n0 = edge_node_indices[0].astype(jnp.int32)
    n1 = edge_node_indices[1].astype(jnp.int32)
    w0T = W1[:, :D_NODE].T
    w1T = W1[:, D_NODE:2 * D_NODE].T
    wcT = W1[:, 2 * D_NODE:].T
    w2aT = W2[:, :D_NODE].T
    w2bT = W2[:, D_NODE:].T
    b1r = b1.reshape(1, MSG)
    b2r = b2.reshape(1, D_NODE)

    P0, P1 = _proj_nodes(x, w0T, w1T)

    efTp = jnp.pad(edge_features.T, ((0, 0), (0, N_EDGES_PAD - N_EDGES)))
    EP = _proj_edges(efTp, wcT, b1r)

    trash = N_NODES + jnp.arange(N_EDGES_PAD - N_EDGES, dtype=jnp.int32) % 16

    def _prep_idx(idx):
        a = jnp.concatenate([idx, trash])
        return a.reshape(N_IDX_ROWS, SUB)

    part = _sc_gather_scatter(P0, P1, EP, _prep_idx(n0), _prep_idx(n1))
    return _final(x, part, w2aT, w2bT, b2r)


# speedup vs baseline: 2.2811x; 1.0282x over previous
import jax
import jax.numpy as jnp
from jax import lax
from jax.experimental import pallas as pl
from jax.experimental.pallas import tpu as pltpu
from jax.experimental.pallas import tpu_sc as plsc

N_NODES = 10000
N_EDGES = 320000
D_NODE = 128
D_EDGE = 16
MSG = 32

NC = 2
NS = 16
NW = NC * NS

C_EDGES = 512
CQ = C_EDGES // 4
SUB = 128
R_CHUNK = C_EDGES // SUB
N_CHUNK = 20
EPW = C_EDGES * N_CHUNK
N_EDGES_PAD = NW * EPW
N_IDX_ROWS = N_EDGES_PAD // SUB
N_NODES_PAD = 10016
NPZ = 624


def _proj_nodes_body(x_ref, w0_ref, w1_ref, p0_ref, p1_ref):
    x = x_ref[...]
    zt = jnp.zeros((N_NODES_PAD - N_NODES, MSG), jnp.float32)
    p0_ref[pl.ds(0, N_NODES), :] = jnp.dot(
        x, w0_ref[...], preferred_element_type=jnp.float32
    )
    p0_ref[pl.ds(N_NODES, N_NODES_PAD - N_NODES), :] = zt
    p1_ref[pl.ds(0, N_NODES), :] = jnp.dot(
        x, w1_ref[...], preferred_element_type=jnp.float32
    )
    p1_ref[pl.ds(N_NODES, N_NODES_PAD - N_NODES), :] = zt


def _proj_nodes(x, w0T, w1T):
    return pl.pallas_call(
        _proj_nodes_body,
        out_shape=[
            jax.ShapeDtypeStruct((N_NODES_PAD, MSG), jnp.float32),
            jax.ShapeDtypeStruct((N_NODES_PAD, MSG), jnp.float32),
        ],
    )(x, w0T, w1T)


E_BLK = 8192


def _proj_edges_body(efT_ref, wc_ref, b1_ref, ep_ref):
    res = (
        lax.dot_general(
            efT_ref[...], wc_ref[...],
            (((0,), (0,)), ((), ())),
            preferred_element_type=jnp.float32,
        )
        + b1_ref[...]
    )
    packed = [
        jnp.concatenate(
            [
                res[C_EDGES * t + CQ * u:C_EDGES * t + CQ * (u + 1)]
                for u in range(4)
            ],
            axis=1,
        )
        for t in range(E_BLK // C_EDGES)
    ]
    ep_ref[...] = jnp.concatenate(packed, axis=0)


def _proj_edges(efTp, wcT, b1r):
    grid = N_EDGES_PAD // E_BLK
    return pl.pallas_call(
        _proj_edges_body,
        grid=(grid,),
        in_specs=[
            pl.BlockSpec((D_EDGE, E_BLK), lambda i: (0, i)),
            pl.BlockSpec((D_EDGE, MSG), lambda i: (0, 0)),
            pl.BlockSpec((1, MSG), lambda i: (0, 0)),
        ],
        out_specs=pl.BlockSpec((E_BLK // 4, 4 * MSG), lambda i: (i, 0)),
        out_shape=jax.ShapeDtypeStruct((N_EDGES_PAD // 4, 4 * MSG), jnp.float32),
    )(efTp, wcT, b1r)


def _sc_body(p0_hbm, p1_hbm, ep_hbm, i0_hbm, i1_hbm, out_hbm,
             i0_v0, i0_v1, i1_v0, i1_v1, ep_v0, ep_v1,
             g0_v0, g0_v1, g1_v0, g1_v1, acc_sh, sem0, sem1):
    cid = lax.axis_index("c")
    sid = lax.axis_index("s")
    wid = sid * NC + cid
    i0_v = (i0_v0, i0_v1)
    i1_v = (i1_v0, i1_v1)
    ep_v = (ep_v0, ep_v1)
    g0_v = (g0_v0, g0_v1)
    g1_v = (g1_v0, g1_v1)
    sem = (sem0, sem1)

    def zrow(r, carry):
        g0_v0[r, pl.ds(0, 16)] = jnp.zeros((16,), jnp.float32)
        g0_v0[r, pl.ds(16, 16)] = jnp.zeros((16,), jnp.float32)
        return carry

    lax.fori_loop(0, C_EDGES, zrow, 0)
    pltpu.sync_copy(
        g0_v0, acc_sh.at[pl.ds(sid * NPZ, C_EDGES)]
    )
    pltpu.sync_copy(
        g0_v0.at[pl.ds(0, NPZ - C_EDGES)],
        acc_sh.at[pl.ds(sid * NPZ + C_EDGES, NPZ - C_EDGES)],
    )

    @pl.when(sid == NS - 1)
    def _zero_tail():
        pltpu.sync_copy(
            g0_v0.at[pl.ds(0, 32)], acc_sh.at[pl.ds(NS * NPZ, 32)]
        )

    plsc.subcore_barrier()

    def _start(slot, jc):
        rbase = jc * R_CHUNK
        pltpu.sync_copy(i0_hbm.at[pl.ds(rbase, R_CHUNK)], i0_v[slot])
        pltpu.sync_copy(i1_hbm.at[pl.ds(rbase, R_CHUNK)], i1_v[slot])
        pltpu.async_copy(ep_hbm.at[pl.ds(jc * CQ, CQ)], ep_v[slot], sem[slot])
        for j in range(R_CHUNK):
            dst = pl.ds(j * SUB, SUB)
            pltpu.async_copy(p0_hbm.at[i0_v[slot].at[j]], g0_v[slot].at[dst],
                             sem[slot])
            pltpu.async_copy(p1_hbm.at[i1_v[slot].at[j]], g1_v[slot].at[dst],
                             sem[slot])

    def _drain(slot, jc):
        pltpu.make_async_copy(
            ep_hbm.at[pl.ds(jc * CQ, CQ)], ep_v[slot], sem[slot]
        ).wait()
        for j in range(R_CHUNK):
            dst = pl.ds(j * SUB, SUB)
            pltpu.make_async_copy(
                p0_hbm.at[i0_v[slot].at[j]], g0_v[slot].at[dst], sem[slot]
            ).wait()
            pltpu.make_async_copy(
                p1_hbm.at[i1_v[slot].at[j]], g1_v[slot].at[dst], sem[slot]
            ).wait()

    def _process(slot, jc):
        _drain(slot, jc)

        @plsc.parallel_loop(0, CQ, unroll=4)
        def rowf(q):
            for u in range(4):
                for off in (0, 16):
                    s = pl.ds(off, 16)
                    se = pl.ds(32 * u + off, 16)
                    g0_v[slot][q + CQ * u, s] = jnp.maximum(
                        g0_v[slot][q + CQ * u, s]
                        + g1_v[slot][q + CQ * u, s]
                        + ep_v[slot][q, se],
                        0.0,
                    )
        for j in range(R_CHUNK):
            pltpu.sync_copy(
                g0_v[slot].at[pl.ds(j * SUB, SUB)],
                acc_sh.at[i0_v[slot].at[j]],
                add=True,
            )

    jc0 = wid * N_CHUNK
    _start(0, jc0)

    def pair(pi, carry):
        jc_a = jc0 + 2 * pi
        _start(1, jc_a + 1)
        _process(0, jc_a)

        @pl.when(pi < N_CHUNK // 2 - 1)
        def _next():
            _start(0, jc_a + 2)

        _process(1, jc_a + 1)
        return carry

    lax.fori_loop(0, N_CHUNK // 2, pair, 0)
    plsc.subcore_barrier()
    pltpu.sync_copy(
        acc_sh.at[pl.ds(sid * NPZ, NPZ)], out_hbm.at[cid, pl.ds(sid * NPZ, NPZ)]
    )

    @pl.when(sid == NS - 1)
    def _write_tail():
        pltpu.sync_copy(
            acc_sh.at[pl.ds(NS * NPZ, 16)], out_hbm.at[cid, pl.ds(NS * NPZ, 16)]
        )


def _sc_gather_scatter(P0, P1, EP, i0, i1):
    mesh = plsc.VectorSubcoreMesh(core_axis_name="c", subcore_axis_name="s")
    return pl.kernel(
        _sc_body,
        out_type=jax.ShapeDtypeStruct((NC, N_NODES, MSG), jnp.float32),
        mesh=mesh,
        compiler_params=pltpu.CompilerParams(use_tc_tiling_on_sc=False),
        scratch_types=[
            pltpu.VMEM((R_CHUNK, SUB), jnp.int32),
            pltpu.VMEM((R_CHUNK, SUB), jnp.int32),
            pltpu.VMEM((R_CHUNK, SUB), jnp.int32),
            pltpu.VMEM((R_CHUNK, SUB), jnp.int32),
            pltpu.VMEM((CQ, 4 * MSG), jnp.float32),
            pltpu.VMEM((CQ, 4 * MSG), jnp.float32),
            pltpu.VMEM((C_EDGES, MSG), jnp.float32),
            pltpu.VMEM((C_EDGES, MSG), jnp.float32),
            pltpu.VMEM((C_EDGES, MSG), jnp.float32),
            pltpu.VMEM((C_EDGES, MSG), jnp.float32),
            pltpu.VMEM_SHARED((N_NODES_PAD, MSG), jnp.float32),
            pltpu.SemaphoreType.DMA,
            pltpu.SemaphoreType.DMA,
        ],
    )(P0, P1, EP, i0, i1)


def _final_body(x_ref, part_ref, w2a_ref, w2b_ref, b2_ref, out_ref):
    acc = part_ref[0] + part_ref[1]
    o = (
        jnp.dot(x_ref[...], w2a_ref[...], preferred_element_type=jnp.float32)
        + jnp.dot(acc, w2b_ref[...], preferred_element_type=jnp.float32)
        + b2_ref[...]
    )
    out_ref[...] = jnp.maximum(o, 0.0)


def _final(x, part, w2aT, w2bT, b2r):
    blk = 1000
    grid = N_NODES // blk
    return pl.pallas_call(
        _final_body,
        grid=(grid,),
        in_specs=[
            pl.BlockSpec((blk, D_NODE), lambda i: (i, 0)),
            pl.BlockSpec((NC, blk, MSG), lambda i: (0, i, 0)),
            pl.BlockSpec((D_NODE, D_NODE), lambda i: (0, 0)),
            pl.BlockSpec((MSG, D_NODE), lambda i: (0, 0)),
            pl.BlockSpec((1, D_NODE), lambda i: (0, 0)),
        ],
        out_specs=pl.BlockSpec((blk, D_NODE), lambda i: (i, 0)),
        out_shape=jax.ShapeDtypeStruct((N_NODES, D_NODE), jnp.float32),
    )(x, part, w2aT, w2bT, b2r)


def kernel(node_features, edge_node_indices, edge_features, W1, b1, W2, b2):
    x = node_features
    n0 = edge_node_indices[0].astype(jnp.int32)
    n1 = edge_node_indices[1].astype(jnp.int32)
    w0T = W1[:, :D_NODE].T
    w1T = W1[:, D_NODE:2 * D_NODE].T
    wcT = W1[:, 2 * D_NODE:].T
    w2aT = W2[:, :D_NODE].T
    w2bT = W2[:, D_NODE:].T
    b1r = b1.reshape(1, MSG)
    b2r = b2.reshape(1, D_NODE)

    P0, P1 = _proj_nodes(x, w0T, w1T)

    efTp = jnp.pad(edge_features.T, ((0, 0), (0, N_EDGES_PAD - N_EDGES)))
    EP = _proj_edges(efTp, wcT, b1r)

    trash = N_NODES + jnp.arange(N_EDGES_PAD - N_EDGES, dtype=jnp.int32) % 16

    def _prep_idx(idx):
        a = jnp.concatenate([idx, trash])
        return a.reshape(N_IDX_ROWS, SUB)

    part = _sc_gather_scatter(P0, P1, EP, _prep_idx(n0), _prep_idx(n1))
    return _final(x, part, w2aT, w2bT, b2r)


# speedup vs baseline: 2.2900x; 1.0039x over previous
import jax
import jax.numpy as jnp
from jax import lax
from jax.experimental import pallas as pl
from jax.experimental.pallas import tpu as pltpu
from jax.experimental.pallas import tpu_sc as plsc

N_NODES = 10000
N_EDGES = 320000
D_NODE = 128
D_EDGE = 16
MSG = 32

NC = 2
NS = 16
NW = NC * NS

C_EDGES = 512
CQ = C_EDGES // 4
SUB = 128
R_CHUNK = C_EDGES // SUB
N_CHUNK = 20
EPW = C_EDGES * N_CHUNK
N_EDGES_PAD = NW * EPW
N_IDX_ROWS = N_EDGES_PAD // SUB
N_NODES_PAD = 10016
NPZ = 624


def _proj_nodes_body(x_ref, w0_ref, w1_ref, p0_ref, p1_ref):
    x = x_ref[...]
    zt = jnp.zeros((N_NODES_PAD - N_NODES, MSG), jnp.float32)
    p0_ref[pl.ds(0, N_NODES), :] = jnp.dot(
        x, w0_ref[...], preferred_element_type=jnp.float32
    )
    p0_ref[pl.ds(N_NODES, N_NODES_PAD - N_NODES), :] = zt
    p1_ref[pl.ds(0, N_NODES), :] = jnp.dot(
        x, w1_ref[...], preferred_element_type=jnp.float32
    )
    p1_ref[pl.ds(N_NODES, N_NODES_PAD - N_NODES), :] = zt


def _proj_nodes(x, w0T, w1T):
    return pl.pallas_call(
        _proj_nodes_body,
        out_shape=[
            jax.ShapeDtypeStruct((N_NODES_PAD, MSG), jnp.float32),
            jax.ShapeDtypeStruct((N_NODES_PAD, MSG), jnp.float32),
        ],
    )(x, w0T, w1T)


E_BLK = 8192


def _proj_edges_body(efT_ref, wc_ref, b1_ref, ep_ref):
    res = (
        lax.dot_general(
            efT_ref[...], wc_ref[...],
            (((0,), (0,)), ((), ())),
            preferred_element_type=jnp.float32,
        )
        + b1_ref[...]
    )
    packed = [
        jnp.concatenate(
            [
                res[C_EDGES * t + CQ * u:C_EDGES * t + CQ * (u + 1)]
                for u in range(4)
            ],
            axis=1,
        )
        for t in range(E_BLK // C_EDGES)
    ]
    ep_ref[...] = jnp.concatenate(packed, axis=0)


def _proj_edges(efTp, wcT, b1r):
    grid = N_EDGES_PAD // E_BLK
    return pl.pallas_call(
        _proj_edges_body,
        grid=(grid,),
        in_specs=[
            pl.BlockSpec((D_EDGE, E_BLK), lambda i: (0, i)),
            pl.BlockSpec((D_EDGE, MSG), lambda i: (0, 0)),
            pl.BlockSpec((1, MSG), lambda i: (0, 0)),
        ],
        out_specs=pl.BlockSpec((E_BLK // 4, 4 * MSG), lambda i: (i, 0)),
        out_shape=jax.ShapeDtypeStruct((N_EDGES_PAD // 4, 4 * MSG), jnp.float32),
    )(efTp, wcT, b1r)


def _sc_body(p0_hbm, p1_hbm, ep_hbm, i0_hbm, i1_hbm, out_hbm,
             i0_v0, i0_v1, i1_v0, i1_v1, ep_v0, ep_v1,
             g0_v0, g0_v1, g1_v0, g1_v1, acc_sh,
             sem0, sem1, sems0, sems1):
    cid = lax.axis_index("c")
    sid = lax.axis_index("s")
    wid = sid * NC + cid
    i0_v = (i0_v0, i0_v1)
    i1_v = (i1_v0, i1_v1)
    ep_v = (ep_v0, ep_v1)
    g0_v = (g0_v0, g0_v1)
    g1_v = (g1_v0, g1_v1)
    sem = (sem0, sem1)
    sems = (sems0, sems1)

    def zrow(r, carry):
        g0_v0[r, pl.ds(0, 16)] = jnp.zeros((16,), jnp.float32)
        g0_v0[r, pl.ds(16, 16)] = jnp.zeros((16,), jnp.float32)
        return carry

    lax.fori_loop(0, C_EDGES, zrow, 0)
    pltpu.sync_copy(
        g0_v0, acc_sh.at[pl.ds(sid * NPZ, C_EDGES)]
    )
    pltpu.sync_copy(
        g0_v0.at[pl.ds(0, NPZ - C_EDGES)],
        acc_sh.at[pl.ds(sid * NPZ + C_EDGES, NPZ - C_EDGES)],
    )

    @pl.when(sid == NS - 1)
    def _zero_tail():
        pltpu.sync_copy(
            g0_v0.at[pl.ds(0, 32)], acc_sh.at[pl.ds(NS * NPZ, 32)]
        )

    plsc.subcore_barrier()

    def _start(slot, jc):
        rbase = jc * R_CHUNK
        pltpu.sync_copy(i0_hbm.at[pl.ds(rbase, R_CHUNK)], i0_v[slot])
        pltpu.sync_copy(i1_hbm.at[pl.ds(rbase, R_CHUNK)], i1_v[slot])
        pltpu.async_copy(ep_hbm.at[pl.ds(jc * CQ, CQ)], ep_v[slot], sem[slot])
        for j in range(R_CHUNK):
            dst = pl.ds(j * SUB, SUB)
            pltpu.async_copy(p0_hbm.at[i0_v[slot].at[j]],
                             g0_v[slot].at[dst], sem[slot])
            pltpu.async_copy(p1_hbm.at[i1_v[slot].at[j]],
                             g1_v[slot].at[dst], sem[slot])

    def _drain_scatters(slot):
        for j in range(R_CHUNK):
            pltpu.make_async_copy(
                g0_v[slot].at[pl.ds(j * SUB, SUB)],
                acc_sh.at[i0_v[slot].at[j]],
                sems[slot],
            ).wait()

    def _process(slot, jc):
        pltpu.make_async_copy(
            ep_hbm.at[pl.ds(jc * CQ, CQ)], ep_v[slot], sem[slot]
        ).wait()
        for j in range(R_CHUNK):
            dst = pl.ds(j * SUB, SUB)
            pltpu.make_async_copy(
                p0_hbm.at[i0_v[slot].at[j]], g0_v[slot].at[dst], sem[slot]
            ).wait()
            pltpu.make_async_copy(
                p1_hbm.at[i1_v[slot].at[j]], g1_v[slot].at[dst], sem[slot]
            ).wait()

        @plsc.parallel_loop(0, CQ, unroll=4)
        def rowf(q):
            for u in range(4):
                for off in (0, 16):
                    s = pl.ds(off, 16)
                    se = pl.ds(32 * u + off, 16)
                    g0_v[slot][q + CQ * u, s] = jnp.maximum(
                        g0_v[slot][q + CQ * u, s]
                        + g1_v[slot][q + CQ * u, s]
                        + ep_v[slot][q, se],
                        0.0,
                    )
        for j in range(R_CHUNK):
            pltpu.async_copy(
                g0_v[slot].at[pl.ds(j * SUB, SUB)],
                acc_sh.at[i0_v[slot].at[j]],
                sems[slot],
                add=True,
            )

    jc0 = wid * N_CHUNK
    _start(0, jc0)

    def pair(pi, carry):
        jc_a = jc0 + 2 * pi

        @pl.when(pi > 0)
        def _ds1():
            _drain_scatters(1)

        _start(1, jc_a + 1)
        _process(0, jc_a)

        @pl.when(pi < N_CHUNK // 2 - 1)
        def _next():
            _drain_scatters(0)
            _start(0, jc_a + 2)

        _process(1, jc_a + 1)
        return carry

    lax.fori_loop(0, N_CHUNK // 2, pair, 0)
    _drain_scatters(0)
    _drain_scatters(1)
    plsc.subcore_barrier()
    pltpu.sync_copy(
        acc_sh.at[pl.ds(sid * NPZ, NPZ)], out_hbm.at[cid, pl.ds(sid * NPZ, NPZ)]
    )

    @pl.when(sid == NS - 1)
    def _write_tail():
        pltpu.sync_copy(
            acc_sh.at[pl.ds(NS * NPZ, 16)], out_hbm.at[cid, pl.ds(NS * NPZ, 16)]
        )


def _sc_gather_scatter(P0, P1, EP, i0, i1):
    mesh = plsc.VectorSubcoreMesh(core_axis_name="c", subcore_axis_name="s")
    return pl.kernel(
        _sc_body,
        out_type=jax.ShapeDtypeStruct((NC, N_NODES, MSG), jnp.float32),
        mesh=mesh,
        compiler_params=pltpu.CompilerParams(use_tc_tiling_on_sc=False),
        scratch_types=[
            pltpu.VMEM((R_CHUNK, SUB), jnp.int32),
            pltpu.VMEM((R_CHUNK, SUB), jnp.int32),
            pltpu.VMEM((R_CHUNK, SUB), jnp.int32),
            pltpu.VMEM((R_CHUNK, SUB), jnp.int32),
            pltpu.VMEM((CQ, 4 * MSG), jnp.float32),
            pltpu.VMEM((CQ, 4 * MSG), jnp.float32),
            pltpu.VMEM((C_EDGES, MSG), jnp.float32),
            pltpu.VMEM((C_EDGES, MSG), jnp.float32),
            pltpu.VMEM((C_EDGES, MSG), jnp.float32),
            pltpu.VMEM((C_EDGES, MSG), jnp.float32),
            pltpu.VMEM_SHARED((N_NODES_PAD, MSG), jnp.float32),
            pltpu.SemaphoreType.DMA,
            pltpu.SemaphoreType.DMA,
            pltpu.SemaphoreType.DMA,
            pltpu.SemaphoreType.DMA,
        ],
    )(P0, P1, EP, i0, i1)


def _final_body(x_ref, part_ref, w2a_ref, w2b_ref, b2_ref, out_ref):
    acc = part_ref[0] + part_ref[1]
    o = (
        jnp.dot(x_ref[...], w2a_ref[...], preferred_element_type=jnp.float32)
        + jnp.dot(acc, w2b_ref[...], preferred_element_type=jnp.float32)
        + b2_ref[...]
    )
    out_ref[...] = jnp.maximum(o, 0.0)


def _final(x, part, w2aT, w2bT, b2r):
    blk = 1000
    grid = N_NODES // blk
    return pl.pallas_call(
        _final_body,
        grid=(grid,),
        in_specs=[
            pl.BlockSpec((blk, D_NODE), lambda i: (i, 0)),
            pl.BlockSpec((NC, blk, MSG), lambda i: (0, i, 0)),
            pl.BlockSpec((D_NODE, D_NODE), lambda i: (0, 0)),
            pl.BlockSpec((MSG, D_NODE), lambda i: (0, 0)),
            pl.BlockSpec((1, D_NODE), lambda i: (0, 0)),
        ],
        out_specs=pl.BlockSpec((blk, D_NODE), lambda i: (i, 0)),
        out_shape=jax.ShapeDtypeStruct((N_NODES, D_NODE), jnp.float32),
    )(x, part, w2aT, w2bT, b2r)


def kernel(node_features, edge_node_indices, edge_features, W1, b1, W2, b2):
    x = node_features
    n0 = edge_node_indices[0].astype(jnp.int32)
    n1 = edge_node_indices[1].astype(jnp.int32)
    w0T = W1[:, :D_NODE].T
    w1T = W1[:, D_NODE:2 * D_NODE].T
    wcT = W1[:, 2 * D_NODE:].T
    w2aT = W2[:, :D_NODE].T
    w2bT = W2[:, D_NODE:].T
    b1r = b1.reshape(1, MSG)
    b2r = b2.reshape(1, D_NODE)

    P0, P1 = _proj_nodes(x, w0T, w1T)

    efTp = jnp.pad(edge_features.T, ((0, 0), (0, N_EDGES_PAD - N_EDGES)))
    EP = _proj_edges(efTp, wcT, b1r)

    trash = N_NODES + jnp.arange(N_EDGES_PAD - N_EDGES, dtype=jnp.int32) % 16

    def _prep_idx(idx):
        a = jnp.concatenate([idx, trash])
        return a.reshape(N_IDX_ROWS, SUB)

    part = _sc_gather_scatter(P0, P1, EP, _prep_idx(n0), _prep_idx(n1))
    return _final(x, part, w2aT, w2bT, b2r)


# speedup vs baseline: 2.4082x; 1.0516x over previous
import jax
import jax.numpy as jnp
from jax import lax
from jax.experimental import pallas as pl
from jax.experimental.pallas import tpu as pltpu
from jax.experimental.pallas import tpu_sc as plsc

N_NODES = 10000
N_EDGES = 320000
D_NODE = 128
D_EDGE = 16
MSG = 32

NC = 2
NS = 16
NW = NC * NS

C_EDGES = 512
CQ = C_EDGES // 4
SUB = 128
R_CHUNK = C_EDGES // SUB
N_CHUNK = 20
EPW = C_EDGES * N_CHUNK
N_EDGES_PAD = NW * EPW
N_IDX_ROWS = N_EDGES_PAD // SUB
N_NODES_PAD = 10016
NPZ = 624


def _proj_nodes_body(x_ref, w0_ref, w1_ref, p0_ref, p1_ref):
    x = x_ref[...]
    zt = jnp.zeros((N_NODES_PAD - N_NODES, MSG), jnp.float32)
    p0_ref[pl.ds(0, N_NODES), :] = jnp.dot(
        x, w0_ref[...], preferred_element_type=jnp.float32
    )
    p0_ref[pl.ds(N_NODES, N_NODES_PAD - N_NODES), :] = zt
    p1_ref[pl.ds(0, N_NODES), :] = jnp.dot(
        x, w1_ref[...], preferred_element_type=jnp.float32
    )
    p1_ref[pl.ds(N_NODES, N_NODES_PAD - N_NODES), :] = zt


def _proj_nodes(x, w0T, w1T):
    return pl.pallas_call(
        _proj_nodes_body,
        out_shape=[
            jax.ShapeDtypeStruct((N_NODES_PAD, MSG), jnp.float32),
            jax.ShapeDtypeStruct((N_NODES_PAD, MSG), jnp.float32),
        ],
    )(x, w0T, w1T)


E_BLK = 8192


def _proj_edges_body(efT_ref, wc_ref, b1_ref, ep_ref):
    res = (
        lax.dot_general(
            efT_ref[...], wc_ref[...],
            (((0,), (0,)), ((), ())),
            preferred_element_type=jnp.float32,
        )
        + b1_ref[...]
    )
    packed = [
        jnp.concatenate(
            [
                res[C_EDGES * t + CQ * u:C_EDGES * t + CQ * (u + 1)]
                for u in range(4)
            ],
            axis=1,
        )
        for t in range(E_BLK // C_EDGES)
    ]
    ep_ref[...] = jnp.concatenate(packed, axis=0)


def _proj_edges(efTp, wcT, b1r):
    grid = N_EDGES_PAD // E_BLK
    return pl.pallas_call(
        _proj_edges_body,
        grid=(grid,),
        in_specs=[
            pl.BlockSpec((D_EDGE, E_BLK), lambda i: (0, i)),
            pl.BlockSpec((D_EDGE, MSG), lambda i: (0, 0)),
            pl.BlockSpec((1, MSG), lambda i: (0, 0)),
        ],
        out_specs=pl.BlockSpec((E_BLK // 4, 4 * MSG), lambda i: (i, 0)),
        out_shape=jax.ShapeDtypeStruct((N_EDGES_PAD // 4, 4 * MSG), jnp.float32),
    )(efTp, wcT, b1r)


def _sc_body(p0_hbm, p1_hbm, ep_hbm, i0_hbm, i1_hbm, out_hbm,
             i0_v0, i0_v1, i1_v0, i1_v1, ep_v0, ep_v1,
             g0_v0, g0_v1, g1_v0, g1_v1, acc_sh,
             sem0, sem1, sems0, sems1):
    cid = lax.axis_index("c")
    sid = lax.axis_index("s")
    wid = sid * NC + cid
    i0_v = (i0_v0, i0_v1)
    i1_v = (i1_v0, i1_v1)
    ep_v = (ep_v0, ep_v1)
    g0_v = (g0_v0, g0_v1)
    g1_v = (g1_v0, g1_v1)
    sem = (sem0, sem1)
    sems = (sems0, sems1)

    def zrow(r, carry):
        g0_v0[r, pl.ds(0, 16)] = jnp.zeros((16,), jnp.float32)
        g0_v0[r, pl.ds(16, 16)] = jnp.zeros((16,), jnp.float32)
        return carry

    lax.fori_loop(0, C_EDGES, zrow, 0)
    pltpu.sync_copy(
        g0_v0, acc_sh.at[pl.ds(sid * NPZ, C_EDGES)]
    )
    pltpu.sync_copy(
        g0_v0.at[pl.ds(0, NPZ - C_EDGES)],
        acc_sh.at[pl.ds(sid * NPZ + C_EDGES, NPZ - C_EDGES)],
    )

    @pl.when(sid == NS - 1)
    def _zero_tail():
        pltpu.sync_copy(
            g0_v0.at[pl.ds(0, 32)], acc_sh.at[pl.ds(NS * NPZ, 32)]
        )

    plsc.subcore_barrier()

    def _start(slot, jc):
        rbase = jc * R_CHUNK
        pltpu.sync_copy(i0_hbm.at[pl.ds(rbase, R_CHUNK)], i0_v[slot])
        pltpu.sync_copy(i1_hbm.at[pl.ds(rbase, R_CHUNK)], i1_v[slot])
        pltpu.async_copy(ep_hbm.at[pl.ds(jc * CQ, CQ)], ep_v[slot], sem[slot])
        for j in range(R_CHUNK):
            dst = pl.ds(j * SUB, SUB)
            pltpu.async_copy(p0_hbm.at[i0_v[slot].at[j]],
                             g0_v[slot].at[dst], sem[slot])
            pltpu.async_copy(p1_hbm.at[i1_v[slot].at[j]],
                             g1_v[slot].at[dst], sem[slot])

    def _drain_scatters(slot):
        for j in range(R_CHUNK):
            pltpu.make_async_copy(
                g0_v[slot].at[pl.ds(j * SUB, SUB)],
                acc_sh.at[i0_v[slot].at[j]],
                sems[slot],
            ).wait()

    def _process(slot, jc):
        pltpu.make_async_copy(
            ep_hbm.at[pl.ds(jc * CQ, CQ)], ep_v[slot], sem[slot]
        ).wait()
        for j in range(R_CHUNK):
            dst = pl.ds(j * SUB, SUB)
            pltpu.make_async_copy(
                p0_hbm.at[i0_v[slot].at[j]], g0_v[slot].at[dst], sem[slot]
            ).wait()
            pltpu.make_async_copy(
                p1_hbm.at[i1_v[slot].at[j]], g1_v[slot].at[dst], sem[slot]
            ).wait()

        @plsc.parallel_loop(0, CQ, unroll=8)
        def rowf(q):
            for u in range(4):
                for off in (0, 16):
                    s = pl.ds(off, 16)
                    se = pl.ds(32 * u + off, 16)
                    g0_v[slot][q + CQ * u, s] = jnp.maximum(
                        g0_v[slot][q + CQ * u, s]
                        + g1_v[slot][q + CQ * u, s]
                        + ep_v[slot][q, se],
                        0.0,
                    )
        for j in range(R_CHUNK):
            pltpu.async_copy(
                g0_v[slot].at[pl.ds(j * SUB, SUB)],
                acc_sh.at[i0_v[slot].at[j]],
                sems[slot],
                add=True,
            )

    jc0 = wid * N_CHUNK
    _start(0, jc0)

    def pair(pi, carry):
        jc_a = jc0 + 2 * pi

        @pl.when(pi > 0)
        def _ds1():
            _drain_scatters(1)

        _start(1, jc_a + 1)
        _process(0, jc_a)

        @pl.when(pi < N_CHUNK // 2 - 1)
        def _next():
            _drain_scatters(0)
            _start(0, jc_a + 2)

        _process(1, jc_a + 1)
        return carry

    lax.fori_loop(0, N_CHUNK // 2, pair, 0)
    _drain_scatters(0)
    _drain_scatters(1)
    plsc.subcore_barrier()
    pltpu.sync_copy(
        acc_sh.at[pl.ds(sid * NPZ, NPZ)], out_hbm.at[cid, pl.ds(sid * NPZ, NPZ)]
    )

    @pl.when(sid == NS - 1)
    def _write_tail():
        pltpu.sync_copy(
            acc_sh.at[pl.ds(NS * NPZ, 16)], out_hbm.at[cid, pl.ds(NS * NPZ, 16)]
        )


def _sc_gather_scatter(P0, P1, EP, i0, i1):
    mesh = plsc.VectorSubcoreMesh(core_axis_name="c", subcore_axis_name="s")
    return pl.kernel(
        _sc_body,
        out_type=jax.ShapeDtypeStruct((NC, N_NODES, MSG), jnp.float32),
        mesh=mesh,
        compiler_params=pltpu.CompilerParams(use_tc_tiling_on_sc=False),
        scratch_types=[
            pltpu.VMEM((R_CHUNK, SUB), jnp.int32),
            pltpu.VMEM((R_CHUNK, SUB), jnp.int32),
            pltpu.VMEM((R_CHUNK, SUB), jnp.int32),
            pltpu.VMEM((R_CHUNK, SUB), jnp.int32),
            pltpu.VMEM((CQ, 4 * MSG), jnp.float32),
            pltpu.VMEM((CQ, 4 * MSG), jnp.float32),
            pltpu.VMEM((C_EDGES, MSG), jnp.float32),
            pltpu.VMEM((C_EDGES, MSG), jnp.float32),
            pltpu.VMEM((C_EDGES, MSG), jnp.float32),
            pltpu.VMEM((C_EDGES, MSG), jnp.float32),
            pltpu.VMEM_SHARED((N_NODES_PAD, MSG), jnp.float32),
            pltpu.SemaphoreType.DMA,
            pltpu.SemaphoreType.DMA,
            pltpu.SemaphoreType.DMA,
            pltpu.SemaphoreType.DMA,
        ],
    )(P0, P1, EP, i0, i1)


def _final_body(x_ref, part_ref, w2a_ref, w2b_ref, b2_ref, out_ref):
    acc = part_ref[0] + part_ref[1]
    o = (
        jnp.dot(x_ref[...], w2a_ref[...], preferred_element_type=jnp.float32)
        + jnp.dot(acc, w2b_ref[...], preferred_element_type=jnp.float32)
        + b2_ref[...]
    )
    out_ref[...] = jnp.maximum(o, 0.0)


def _final(x, part, w2aT, w2bT, b2r):
    blk = 1000
    grid = N_NODES // blk
    return pl.pallas_call(
        _final_body,
        grid=(grid,),
        in_specs=[
            pl.BlockSpec((blk, D_NODE), lambda i: (i, 0)),
            pl.BlockSpec((NC, blk, MSG), lambda i: (0, i, 0)),
            pl.BlockSpec((D_NODE, D_NODE), lambda i: (0, 0)),
            pl.BlockSpec((MSG, D_NODE), lambda i: (0, 0)),
            pl.BlockSpec((1, D_NODE), lambda i: (0, 0)),
        ],
        out_specs=pl.BlockSpec((blk, D_NODE), lambda i: (i, 0)),
        out_shape=jax.ShapeDtypeStruct((N_NODES, D_NODE), jnp.float32),
    )(x, part, w2aT, w2bT, b2r)


def kernel(node_features, edge_node_indices, edge_features, W1, b1, W2, b2):
    x = node_features
    n0 = edge_node_indices[0].astype(jnp.int32)
    n1 = edge_node_indices[1].astype(jnp.int32)
    w0T = W1[:, :D_NODE].T
    w1T = W1[:, D_NODE:2 * D_NODE].T
    wcT = W1[:, 2 * D_NODE:].T
    w2aT = W2[:, :D_NODE].T
    w2bT = W2[:, D_NODE:].T
    b1r = b1.reshape(1, MSG)
    b2r = b2.reshape(1, D_NODE)

    P0, P1 = _proj_nodes(x, w0T, w1T)

    EP = _proj_edges(edge_features.T, wcT, b1r)

    trash = N_NODES + jnp.arange(N_EDGES_PAD - N_EDGES, dtype=jnp.int32) % 16

    def _prep_idx(idx):
        a = jnp.concatenate([idx, trash])
        return a.reshape(N_IDX_ROWS, SUB)

    part = _sc_gather_scatter(P0, P1, EP, _prep_idx(n0), _prep_idx(n1))
    return _final(x, part, w2aT, w2bT, b2r)


# speedup vs baseline: 2.4224x; 1.0059x over previous
import jax
import jax.numpy as jnp
from jax import lax
from jax.experimental import pallas as pl
from jax.experimental.pallas import tpu as pltpu
from jax.experimental.pallas import tpu_sc as plsc

N_NODES = 10000
N_EDGES = 320000
D_NODE = 128
D_EDGE = 16
MSG = 32

NC = 2
NS = 16
NW = NC * NS

C_EDGES = 512
CQ = C_EDGES // 4
SUB = 128
R_CHUNK = C_EDGES // SUB
N_CHUNK = 20
EPW = C_EDGES * N_CHUNK
N_EDGES_PAD = NW * EPW
N_IDX_ROWS = N_EDGES_PAD // SUB
N_NODES_PAD = 10016
NPZ = 624


def _proj_nodes_body(x_ref, w0_ref, w1_ref, p0_ref, p1_ref):
    x = x_ref[...]
    zt = jnp.zeros((N_NODES_PAD - N_NODES, MSG), jnp.float32)
    p0_ref[pl.ds(0, N_NODES), :] = jnp.dot(
        x, w0_ref[...], preferred_element_type=jnp.float32
    )
    p0_ref[pl.ds(N_NODES, N_NODES_PAD - N_NODES), :] = zt
    p1_ref[pl.ds(0, N_NODES), :] = jnp.dot(
        x, w1_ref[...], preferred_element_type=jnp.float32
    )
    p1_ref[pl.ds(N_NODES, N_NODES_PAD - N_NODES), :] = zt


def _proj_nodes(x, w0T, w1T):
    return pl.pallas_call(
        _proj_nodes_body,
        out_shape=[
            jax.ShapeDtypeStruct((N_NODES_PAD, MSG), jnp.float32),
            jax.ShapeDtypeStruct((N_NODES_PAD, MSG), jnp.float32),
        ],
    )(x, w0T, w1T)


E_BLK = 16384


def _proj_edges_body(efT_ref, wc_ref, b1_ref, ep_ref):
    res = (
        lax.dot_general(
            efT_ref[...], wc_ref[...],
            (((0,), (0,)), ((), ())),
            preferred_element_type=jnp.float32,
        )
        + b1_ref[...]
    )
    packed = [
        jnp.concatenate(
            [
                res[C_EDGES * t + CQ * u:C_EDGES * t + CQ * (u + 1)]
                for u in range(4)
            ],
            axis=1,
        )
        for t in range(E_BLK // C_EDGES)
    ]
    ep_ref[...] = jnp.concatenate(packed, axis=0)


def _proj_edges(efTp, wcT, b1r):
    grid = N_EDGES_PAD // E_BLK
    return pl.pallas_call(
        _proj_edges_body,
        grid=(grid,),
        in_specs=[
            pl.BlockSpec((D_EDGE, E_BLK), lambda i: (0, i)),
            pl.BlockSpec((D_EDGE, MSG), lambda i: (0, 0)),
            pl.BlockSpec((1, MSG), lambda i: (0, 0)),
        ],
        out_specs=pl.BlockSpec((E_BLK // 4, 4 * MSG), lambda i: (i, 0)),
        out_shape=jax.ShapeDtypeStruct((N_EDGES_PAD // 4, 4 * MSG), jnp.float32),
    )(efTp, wcT, b1r)


def _sc_body(p0_hbm, p1_hbm, ep_hbm, i0_hbm, i1_hbm, out_hbm,
             i0_v0, i0_v1, i1_v0, i1_v1, ep_v0, ep_v1,
             g0_v0, g0_v1, g1_v0, g1_v1, acc_sh,
             sem0, sem1, sems0, sems1):
    cid = lax.axis_index("c")
    sid = lax.axis_index("s")
    wid = sid * NC + cid
    i0_v = (i0_v0, i0_v1)
    i1_v = (i1_v0, i1_v1)
    ep_v = (ep_v0, ep_v1)
    g0_v = (g0_v0, g0_v1)
    g1_v = (g1_v0, g1_v1)
    sem = (sem0, sem1)
    sems = (sems0, sems1)

    def zrow(r, carry):
        g0_v0[r, pl.ds(0, 16)] = jnp.zeros((16,), jnp.float32)
        g0_v0[r, pl.ds(16, 16)] = jnp.zeros((16,), jnp.float32)
        return carry

    lax.fori_loop(0, C_EDGES, zrow, 0)
    pltpu.sync_copy(
        g0_v0, acc_sh.at[pl.ds(sid * NPZ, C_EDGES)]
    )
    pltpu.sync_copy(
        g0_v0.at[pl.ds(0, NPZ - C_EDGES)],
        acc_sh.at[pl.ds(sid * NPZ + C_EDGES, NPZ - C_EDGES)],
    )

    @pl.when(sid == NS - 1)
    def _zero_tail():
        pltpu.sync_copy(
            g0_v0.at[pl.ds(0, 32)], acc_sh.at[pl.ds(NS * NPZ, 32)]
        )

    plsc.subcore_barrier()

    def _start(slot, jc):
        rbase = jc * R_CHUNK
        pltpu.sync_copy(i0_hbm.at[pl.ds(rbase, R_CHUNK)], i0_v[slot])
        pltpu.sync_copy(i1_hbm.at[pl.ds(rbase, R_CHUNK)], i1_v[slot])
        pltpu.async_copy(ep_hbm.at[pl.ds(jc * CQ, CQ)], ep_v[slot], sem[slot])
        for j in range(R_CHUNK):
            dst = pl.ds(j * SUB, SUB)
            pltpu.async_copy(p0_hbm.at[i0_v[slot].at[j]],
                             g0_v[slot].at[dst], sem[slot])
            pltpu.async_copy(p1_hbm.at[i1_v[slot].at[j]],
                             g1_v[slot].at[dst], sem[slot])

    def _drain_scatters(slot):
        for j in range(R_CHUNK):
            pltpu.make_async_copy(
                g0_v[slot].at[pl.ds(j * SUB, SUB)],
                acc_sh.at[i0_v[slot].at[j]],
                sems[slot],
            ).wait()

    def _process(slot, jc):
        pltpu.make_async_copy(
            ep_hbm.at[pl.ds(jc * CQ, CQ)], ep_v[slot], sem[slot]
        ).wait()
        for j in range(R_CHUNK):
            dst = pl.ds(j * SUB, SUB)
            pltpu.make_async_copy(
                p0_hbm.at[i0_v[slot].at[j]], g0_v[slot].at[dst], sem[slot]
            ).wait()
            pltpu.make_async_copy(
                p1_hbm.at[i1_v[slot].at[j]], g1_v[slot].at[dst], sem[slot]
            ).wait()

        @plsc.parallel_loop(0, CQ, unroll=8)
        def rowf(q):
            for u in range(4):
                for off in (0, 16):
                    s = pl.ds(off, 16)
                    se = pl.ds(32 * u + off, 16)
                    g0_v[slot][q + CQ * u, s] = jnp.maximum(
                        g0_v[slot][q + CQ * u, s]
                        + g1_v[slot][q + CQ * u, s]
                        + ep_v[slot][q, se],
                        0.0,
                    )
        for j in range(R_CHUNK):
            pltpu.async_copy(
                g0_v[slot].at[pl.ds(j * SUB, SUB)],
                acc_sh.at[i0_v[slot].at[j]],
                sems[slot],
                add=True,
            )

    jc0 = wid * N_CHUNK
    _start(0, jc0)

    def pair(pi, carry):
        jc_a = jc0 + 2 * pi

        @pl.when(pi > 0)
        def _ds1():
            _drain_scatters(1)

        _start(1, jc_a + 1)
        _process(0, jc_a)

        @pl.when(pi < N_CHUNK // 2 - 1)
        def _next():
            _drain_scatters(0)
            _start(0, jc_a + 2)

        _process(1, jc_a + 1)
        return carry

    lax.fori_loop(0, N_CHUNK // 2, pair, 0)
    _drain_scatters(0)
    _drain_scatters(1)
    plsc.subcore_barrier()
    pltpu.sync_copy(
        acc_sh.at[pl.ds(sid * NPZ, NPZ)], out_hbm.at[cid, pl.ds(sid * NPZ, NPZ)]
    )

    @pl.when(sid == NS - 1)
    def _write_tail():
        pltpu.sync_copy(
            acc_sh.at[pl.ds(NS * NPZ, 16)], out_hbm.at[cid, pl.ds(NS * NPZ, 16)]
        )


def _sc_gather_scatter(P0, P1, EP, i0, i1):
    mesh = plsc.VectorSubcoreMesh(core_axis_name="c", subcore_axis_name="s")
    return pl.kernel(
        _sc_body,
        out_type=jax.ShapeDtypeStruct((NC, N_NODES, MSG), jnp.float32),
        mesh=mesh,
        compiler_params=pltpu.CompilerParams(use_tc_tiling_on_sc=False),
        scratch_types=[
            pltpu.VMEM((R_CHUNK, SUB), jnp.int32),
            pltpu.VMEM((R_CHUNK, SUB), jnp.int32),
            pltpu.VMEM((R_CHUNK, SUB), jnp.int32),
            pltpu.VMEM((R_CHUNK, SUB), jnp.int32),
            pltpu.VMEM((CQ, 4 * MSG), jnp.float32),
            pltpu.VMEM((CQ, 4 * MSG), jnp.float32),
            pltpu.VMEM((C_EDGES, MSG), jnp.float32),
            pltpu.VMEM((C_EDGES, MSG), jnp.float32),
            pltpu.VMEM((C_EDGES, MSG), jnp.float32),
            pltpu.VMEM((C_EDGES, MSG), jnp.float32),
            pltpu.VMEM_SHARED((N_NODES_PAD, MSG), jnp.float32),
            pltpu.SemaphoreType.DMA,
            pltpu.SemaphoreType.DMA,
            pltpu.SemaphoreType.DMA,
            pltpu.SemaphoreType.DMA,
        ],
    )(P0, P1, EP, i0, i1)


def _final_body(x_ref, part_ref, w2a_ref, w2b_ref, b2_ref, out_ref):
    acc = part_ref[0] + part_ref[1]
    o = (
        jnp.dot(x_ref[...], w2a_ref[...], preferred_element_type=jnp.float32)
        + jnp.dot(acc, w2b_ref[...], preferred_element_type=jnp.float32)
        + b2_ref[...]
    )
    out_ref[...] = jnp.maximum(o, 0.0)


def _final(x, part, w2aT, w2bT, b2r):
    blk = 1000
    grid = N_NODES // blk
    return pl.pallas_call(
        _final_body,
        grid=(grid,),
        in_specs=[
            pl.BlockSpec((blk, D_NODE), lambda i: (i, 0)),
            pl.BlockSpec((NC, blk, MSG), lambda i: (0, i, 0)),
            pl.BlockSpec((D_NODE, D_NODE), lambda i: (0, 0)),
            pl.BlockSpec((MSG, D_NODE), lambda i: (0, 0)),
            pl.BlockSpec((1, D_NODE), lambda i: (0, 0)),
        ],
        out_specs=pl.BlockSpec((blk, D_NODE), lambda i: (i, 0)),
        out_shape=jax.ShapeDtypeStruct((N_NODES, D_NODE), jnp.float32),
    )(x, part, w2aT, w2bT, b2r)


def kernel(node_features, edge_node_indices, edge_features, W1, b1, W2, b2):
    x = node_features
    n0 = edge_node_indices[0].astype(jnp.int32)
    n1 = edge_node_indices[1].astype(jnp.int32)
    w0T = W1[:, :D_NODE].T
    w1T = W1[:, D_NODE:2 * D_NODE].T
    wcT = W1[:, 2 * D_NODE:].T
    w2aT = W2[:, :D_NODE].T
    w2bT = W2[:, D_NODE:].T
    b1r = b1.reshape(1, MSG)
    b2r = b2.reshape(1, D_NODE)

    P0, P1 = _proj_nodes(x, w0T, w1T)

    EP = _proj_edges(edge_features.T, wcT, b1r)

    trash = N_NODES + jnp.arange(N_EDGES_PAD - N_EDGES, dtype=jnp.int32) % 16

    def _prep_idx(idx):
        a = jnp.concatenate([idx, trash])
        return a.reshape(N_IDX_ROWS, SUB)

    part = _sc_gather_scatter(P0, P1, EP, _prep_idx(n0), _prep_idx(n1))
    return _final(x, part, w2aT, w2bT, b2r)
